# baseline (speedup 1.0000x reference)
import jax
import jax.numpy as jnp
from jax import lax
from jax.experimental import pallas as pl
from jax.experimental.pallas import tpu as pltpu
from jax.experimental.pallas import tpu_sc as plsc

B = 4
H = 8192
F = 1024
NC, NS = 2, 16
NW = NC * NS
ROWS_W = H // NW
CR = 8
NCH = ROWS_W // CR
VEC = 16
NV = F // VEC


def _sc_body(x_hbm, w_hbm, o_hbm,
             w0, x00, x01, x02, x03,
             w1, x10, x11, x12, x13,
             sem_in0, sem_in1, sem_out0, sem_out1):
    wid = lax.axis_index("s") * NC + lax.axis_index("c")
    base = wid * ROWS_W

    sets = ((w0, (x00, x01, x02, x03), sem_in0, sem_out0),
            (w1, (x10, x11, x12, x13), sem_in1, sem_out1))

    def fire_in(p, c):
        wbuf, xbufs, sem_in, _ = sets[p]
        r0 = base + c * CR
        pltpu.async_copy(w_hbm.at[pl.ds(r0, CR), :], wbuf, sem_in)
        for b in range(B):
            pltpu.async_copy(x_hbm.at[b, pl.ds(r0, CR), :], xbufs[b], sem_in)

    def drain_in(p, c):
        wbuf, xbufs, sem_in, _ = sets[p]
        r0 = base + c * CR
        pltpu.make_async_copy(w_hbm.at[pl.ds(r0, CR), :], wbuf, sem_in).wait()
        for b in range(B):
            pltpu.make_async_copy(
                x_hbm.at[b, pl.ds(r0, CR), :], xbufs[b], sem_in).wait()

    def drain_out(p, c):
        _, xbufs, _, sem_out = sets[p]
        r0 = base + c * CR
        for b in range(B):
            pltpu.make_async_copy(
                xbufs[b], o_hbm.at[b, pl.ds(r0, CR), :], sem_out).wait()

    fire_in(0, 0)

    def chunk_pair(c2, carry):
        for p in (0, 1):
            wbuf, xbufs, _, sem_out = sets[p]
            c = 2 * c2 + p
            r0 = base + c * CR

            @pl.when(c >= 1)
            def _():
                drain_out(1 - p, c - 1)

            @pl.when(c + 1 < NCH)
            def _():
                fire_in(1 - p, c + 1)

            drain_in(p, c)

            for b in range(B):
                xb = xbufs[b]

                def vec_loop(i, xb=xb):
                    s = pl.ds(i * VEC, VEC)
                    for r in range(CR):
                        xb[r, s] = xb[r, s] + wbuf[r, s]

                plsc.parallel_loop(0, NV, unroll=8)(vec_loop)
                pltpu.async_copy(xb, o_hbm.at[b, pl.ds(r0, CR), :], sem_out)
        return carry

    lax.fori_loop(0, NCH // 2, chunk_pair, 0)
    drain_out((NCH - 1) % 2, NCH - 1)


@jax.jit
def kernel(x, embed_weight):
    mesh = plsc.VectorSubcoreMesh(core_axis_name="c", subcore_axis_name="s")
    return pl.kernel(
        _sc_body,
        out_type=jax.ShapeDtypeStruct((B, H, F), jnp.float32),
        mesh=mesh,
        scratch_types=(
            [pltpu.VMEM((CR, F), jnp.float32)] * 10
            + [pltpu.SemaphoreType.DMA] * 4
        ),
    )(x, embed_weight)


# speedup vs baseline: 1.0223x; 1.0223x over previous
import jax
import jax.numpy as jnp
from jax import lax
from jax.experimental import pallas as pl
from jax.experimental.pallas import tpu as pltpu
from jax.experimental.pallas import tpu_sc as plsc

B = 4
H = 8192
F = 1024
NC, NS = 2, 16
NW = NC * NS
ROWS_W = H // NW
CR = 16
NCH = ROWS_W // CR
NSTEP = NCH * B
VEC = 16
NV = F // VEC


def _sc_body(x_hbm, w_hbm, o_hbm,
             w0, w1, xr0, xr1, xr2, xr3,
             semw0, semw1, semx0, semx1, semx2, semx3,
             semo0, semo1, semo2, semo3):
    wid = lax.axis_index("s") * NC + lax.axis_index("c")
    base = wid * ROWS_W

    wbufs = (w0, w1)
    semw = (semw0, semw1)
    xring = (xr0, xr1, xr2, xr3)
    semx = (semx0, semx1, semx2, semx3)
    semo = (semo0, semo1, semo2, semo3)

    def fire_w(p, c):
        pltpu.async_copy(
            w_hbm.at[pl.ds(base + c * CR, CR), :], wbufs[p], semw[p])

    def wait_w(p, c):
        pltpu.make_async_copy(
            w_hbm.at[pl.ds(base + c * CR, CR), :], wbufs[p], semw[p]).wait()

    def fire_in(sl, c, b):
        pltpu.async_copy(
            x_hbm.at[b, pl.ds(base + c * CR, CR), :], xring[sl], semx[sl])

    def wait_in(sl, c, b):
        pltpu.make_async_copy(
            x_hbm.at[b, pl.ds(base + c * CR, CR), :], xring[sl],
            semx[sl]).wait()

    def fire_out(sl, c, b):
        pltpu.async_copy(
            xring[sl], o_hbm.at[b, pl.ds(base + c * CR, CR), :], semo[sl])

    def wait_out(sl, c, b):
        pltpu.make_async_copy(
            xring[sl], o_hbm.at[b, pl.ds(base + c * CR, CR), :],
            semo[sl]).wait()

    fire_w(0, 0)
    fire_w(1, 1)
    fire_in(0, 0, 0)
    fire_in(1, 0, 1)

    def chunk_pair(c2, carry):
        for p in (0, 1):
            c = 2 * c2 + p
            wait_w(p, c)
            for b in range(B):
                sl2 = (b + 2) % 4
                if b >= 2:
                    wait_out(sl2, c, b - 2)

                    @pl.when(c + 1 < NCH)
                    def _(sl2=sl2, c=c, b=b):
                        fire_in(sl2, c + 1, b - 2)
                else:
                    @pl.when(c >= 1)
                    def _(sl2=sl2, c=c, b=b):
                        wait_out(sl2, c - 1, b + 2)
                    fire_in(sl2, c, b + 2)

                wait_in(b, c, b)
                xb = xring[b]
                wb = wbufs[p]

                def vec_loop(i, xb=xb, wb=wb):
                    s = pl.ds(i * VEC, VEC)
                    for r in range(CR):
                        xb[r, s] = xb[r, s] + wb[r, s]

                plsc.parallel_loop(0, NV, unroll=4)(vec_loop)
                fire_out(b, c, b)

            @pl.when(c + 2 < NCH)
            def _(p=p, c=c):
                fire_w(p, c + 2)
        return carry

    lax.fori_loop(0, NCH // 2, chunk_pair, 0)
    wait_out(2, NCH - 1, 2)
    wait_out(3, NCH - 1, 3)


@jax.jit
def kernel(x, embed_weight):
    mesh = plsc.VectorSubcoreMesh(core_axis_name="c", subcore_axis_name="s")
    return pl.kernel(
        _sc_body,
        out_type=jax.ShapeDtypeStruct((B, H, F), jnp.float32),
        mesh=mesh,
        scratch_types=(
            [pltpu.VMEM((CR, F), jnp.float32)] * 6
            + [pltpu.SemaphoreType.DMA] * 10
        ),
    )(x, embed_weight)
